# baseline (speedup 1.0000x reference)
import functools

import jax
import jax.numpy as jnp
from jax.experimental import pallas as pl

B = 1024
NOBJ = 11
V = 4000
VSUB = 600
NBB = 8
NKP = 16
KC = 72


def _w3_builder_kernel(ang_ref, go_ref, tr_ref, qi_ref, w_ref):
    a = ang_ref[0:1, :]
    ca = jnp.cos(a * 0.5)
    sa = jnp.sin(a * 0.5)
    gx = go_ref[0:1, :]
    gy = go_ref[1:2, :]
    gz = go_ref[2:3, :]
    ang = jnp.sqrt(gx * gx + gy * gy + gz * gz)
    half = ang * 0.5
    small = jnp.abs(ang) < 1e-6
    safe = jnp.where(small, jnp.ones_like(ang), ang)
    sho = jnp.where(small, 0.5 - ang * ang / 48.0, jnp.sin(half) / safe)
    qw = jnp.cos(half)
    qx = gx * sho
    qy = gy * sho
    qz = gz * sho
    tw = qw * ca + qz * sa
    tx = qx * ca - qy * sa
    ty = qy * ca + qx * sa
    tz = qz * ca - qw * sa

    def mat(w, x, y, z):
        return [[1 - 2 * (y * y + z * z), 2 * (x * y - w * z), 2 * (x * z + w * y)],
                [2 * (x * y + w * z), 1 - 2 * (x * x + z * z), 2 * (y * z - w * x)],
                [2 * (x * z - w * y), 2 * (y * z + w * x), 1 - 2 * (x * x + y * y)]]

    mt = mat(tw, tx, ty, tz)
    mb = mat(qw, qx, qy, qz)
    qi = qi_ref[0:1, :]
    oids = jax.lax.broadcasted_iota(jnp.int32, (NOBJ, B), 0)
    onehot = (oids == qi).astype(jnp.float32)
    zrow = jnp.zeros((KC - 67, B), jnp.float32)
    for p in range(3):
        r = p * KC
        for k in range(3):
            w_ref[r + k * NOBJ:r + (k + 1) * NOBJ, :] = onehot * mt[p][k]
            w_ref[r + 34 + k * NOBJ:r + 34 + (k + 1) * NOBJ, :] = onehot * mb[p][k]
        w_ref[r + 33:r + 34, :] = tr_ref[p:p + 1, :]
        w_ref[r + 67:r + KC, :] = zrow


def _mm_kernel(w_ref, xv_ref, xvs_ref, xbb_ref, xkp_ref,
               ov_ref, ovs_ref, obb_ref, okp_ref):
    w = w_ref[...]
    f32 = jnp.float32
    ov_ref[...] = jnp.dot(xv_ref[...], w, preferred_element_type=f32)
    ovs_ref[...] = jnp.dot(xvs_ref[...], w, preferred_element_type=f32)
    obb_ref[...] = jnp.dot(xbb_ref[...], w, preferred_element_type=f32)
    okp_ref[...] = jnp.dot(xkp_ref[...], w, preferred_element_type=f32)


def _x_masked(tab, mask_top, n):
    mt = mask_top.astype(jnp.float32)[..., None]
    t33 = (tab * mt).transpose(1, 2, 0).reshape(n, 3 * NOBJ)
    b33 = (tab * (1.0 - mt)).transpose(1, 2, 0).reshape(n, 3 * NOBJ)
    ones = jnp.ones((n, 1), jnp.float32)
    pad = jnp.zeros((n, KC - 67), jnp.float32)
    return jnp.concatenate([t33, ones, b33, pad], axis=1)


def _x_pair(tab_top, tab_bot, n):
    t33 = tab_top.transpose(1, 2, 0).reshape(n, 3 * NOBJ)
    b33 = tab_bot.transpose(1, 2, 0).reshape(n, 3 * NOBJ)
    z = jnp.zeros((n, 3 * NOBJ), jnp.float32)
    ones = jnp.ones((n, 1), jnp.float32)
    pad = jnp.zeros((n, KC - 67), jnp.float32)
    top_rows = jnp.concatenate([t33, ones, z, pad], axis=1)
    bot_rows = jnp.concatenate([z, ones, b33, pad], axis=1)
    return jnp.concatenate([top_rows, bot_rows], axis=0)


@functools.partial(jax.jit, static_argnames=())
def kernel(angles, global_orient, transl, query_idx, v, v_sub, bbox_top,
           bbox_bottom, kp_top, kp_bottom, parts_ids, parts_sub_ids):
    ang_t = angles.reshape(B, 1).T
    go_t = global_orient.T
    tr_t = transl.T
    qi_t = query_idx.astype(jnp.int32).reshape(B, 1).T

    w3 = pl.pallas_call(
        _w3_builder_kernel,
        out_shape=jax.ShapeDtypeStruct((3 * KC, B), jnp.float32),
    )(ang_t, go_t, tr_t, qi_t)

    xv = _x_masked(v, parts_ids == 1, V)
    xvs = _x_masked(v_sub, parts_sub_ids == 1, VSUB)
    xbb = _x_pair(bbox_top, bbox_bottom, NBB)
    xkp = _x_pair(kp_top, kp_bottom, NKP)

    bn = 256
    grid = (3, B // bn)
    ov, ovs, obb, okp = pl.pallas_call(
        _mm_kernel,
        grid=grid,
        in_specs=[
            pl.BlockSpec((KC, bn), lambda p, b: (p, b)),
            pl.BlockSpec((V, KC), lambda p, b: (0, 0)),
            pl.BlockSpec((VSUB, KC), lambda p, b: (0, 0)),
            pl.BlockSpec((2 * NBB, KC), lambda p, b: (0, 0)),
            pl.BlockSpec((2 * NKP, KC), lambda p, b: (0, 0)),
        ],
        out_specs=[
            pl.BlockSpec((V, bn), lambda p, b: (p, b)),
            pl.BlockSpec((VSUB, bn), lambda p, b: (p, b)),
            pl.BlockSpec((2 * NBB, bn), lambda p, b: (p, b)),
            pl.BlockSpec((2 * NKP, bn), lambda p, b: (p, b)),
        ],
        out_shape=[
            jax.ShapeDtypeStruct((3 * V, B), jnp.float32),
            jax.ShapeDtypeStruct((3 * VSUB, B), jnp.float32),
            jax.ShapeDtypeStruct((3 * 2 * NBB, B), jnp.float32),
            jax.ShapeDtypeStruct((3 * 2 * NKP, B), jnp.float32),
        ],
    )(w3, xv, xvs, xbb, xkp)

    v_out = jnp.transpose(ov.reshape(3, V, B), (2, 1, 0))
    vs_out = jnp.transpose(ovs.reshape(3, VSUB, B), (2, 1, 0))
    bbox3d = jnp.transpose(obb.reshape(3, 2 * NBB, B), (2, 1, 0))
    kp3d = jnp.transpose(okp.reshape(3, 2 * NKP, B), (2, 1, 0))
    return v_out, vs_out, bbox3d, kp3d


# speedup vs baseline: 1.1054x; 1.1054x over previous
import functools

import jax
import jax.numpy as jnp
from jax.experimental import pallas as pl

B = 1024
NOBJ = 11
V = 4000
VSUB = 600
NBB = 8
NKP = 16
KC = 72


def _w3_builder_kernel(ang_ref, go_ref, tr_ref, qi_ref, w_ref):
    a = ang_ref[0:1, :]
    ca = jnp.cos(a * 0.5)
    sa = jnp.sin(a * 0.5)
    gx = go_ref[0:1, :]
    gy = go_ref[1:2, :]
    gz = go_ref[2:3, :]
    ang = jnp.sqrt(gx * gx + gy * gy + gz * gz)
    half = ang * 0.5
    small = jnp.abs(ang) < 1e-6
    safe = jnp.where(small, jnp.ones_like(ang), ang)
    sho = jnp.where(small, 0.5 - ang * ang / 48.0, jnp.sin(half) / safe)
    qw = jnp.cos(half)
    qx = gx * sho
    qy = gy * sho
    qz = gz * sho
    tw = qw * ca + qz * sa
    tx = qx * ca - qy * sa
    ty = qy * ca + qx * sa
    tz = qz * ca - qw * sa

    def mat(w, x, y, z):
        return [[1 - 2 * (y * y + z * z), 2 * (x * y - w * z), 2 * (x * z + w * y)],
                [2 * (x * y + w * z), 1 - 2 * (x * x + z * z), 2 * (y * z - w * x)],
                [2 * (x * z - w * y), 2 * (y * z + w * x), 1 - 2 * (x * x + y * y)]]

    mt = mat(tw, tx, ty, tz)
    mb = mat(qw, qx, qy, qz)
    qi = qi_ref[0:1, :]
    oids = jax.lax.broadcasted_iota(jnp.int32, (NOBJ, B), 0)
    onehot = (oids == qi).astype(jnp.float32)
    zrow = jnp.zeros((KC - 67, B), jnp.float32)
    ohr = [onehot[o:o + 1, :] for o in range(NOBJ)]
    for p in range(3):
        r = p * KC
        top33 = jnp.concatenate(
            [mt[p][k] * ohr[o] for o in range(NOBJ) for k in range(3)], axis=0)
        bot33 = jnp.concatenate(
            [mb[p][k] * ohr[o] for o in range(NOBJ) for k in range(3)], axis=0)
        w_ref[r:r + 33, :] = top33
        w_ref[r + 33:r + 34, :] = tr_ref[p:p + 1, :]
        w_ref[r + 34:r + 67, :] = bot33
        w_ref[r + 67:r + KC, :] = zrow


_DN_T = (((0,), (0,)), ((), ()))


def _mm_kernel(w_ref, xv_ref, xvs_ref, xbb_ref, xkp_ref,
               ov_ref, ovs_ref, obb_ref, okp_ref):
    w = w_ref[...]
    f32 = jnp.float32
    ov_ref[...] = jax.lax.dot_general(
        xv_ref[...], w, _DN_T, preferred_element_type=f32)
    ovs_ref[...] = jax.lax.dot_general(
        xvs_ref[...], w, _DN_T, preferred_element_type=f32)
    obb_ref[...] = jax.lax.dot_general(
        xbb_ref[...], w, _DN_T, preferred_element_type=f32)
    okp_ref[...] = jax.lax.dot_general(
        xkp_ref[...], w, _DN_T, preferred_element_type=f32)


def _x_masked(tab, mask_top, n):
    vp = tab.transpose(0, 2, 1).reshape(3 * NOBJ, n)
    mx = jnp.repeat(mask_top.astype(jnp.float32), 3, axis=0)
    return jnp.concatenate([
        vp * mx,
        jnp.ones((1, n), jnp.float32),
        vp * (1.0 - mx),
        jnp.zeros((KC - 67, n), jnp.float32),
    ], axis=0)


def _x_pair(tab_top, tab_bot, n):
    tp = tab_top.transpose(0, 2, 1).reshape(3 * NOBJ, n)
    bp = tab_bot.transpose(0, 2, 1).reshape(3 * NOBJ, n)
    z = jnp.zeros((3 * NOBJ, n), jnp.float32)
    return jnp.concatenate([
        jnp.concatenate([tp, z], axis=1),
        jnp.ones((1, 2 * n), jnp.float32),
        jnp.concatenate([z, bp], axis=1),
        jnp.zeros((KC - 67, 2 * n), jnp.float32),
    ], axis=0)


@functools.partial(jax.jit, static_argnames=())
def kernel(angles, global_orient, transl, query_idx, v, v_sub, bbox_top,
           bbox_bottom, kp_top, kp_bottom, parts_ids, parts_sub_ids):
    ang_t = angles.reshape(B, 1).T
    go_t = global_orient.T
    tr_t = transl.T
    qi_t = query_idx.astype(jnp.int32).reshape(B, 1).T

    w3 = pl.pallas_call(
        _w3_builder_kernel,
        out_shape=jax.ShapeDtypeStruct((3 * KC, B), jnp.float32),
    )(ang_t, go_t, tr_t, qi_t)

    xv = _x_masked(v, parts_ids == 1, V)
    xvs = _x_masked(v_sub, parts_sub_ids == 1, VSUB)
    xbb = _x_pair(bbox_top, bbox_bottom, NBB)
    xkp = _x_pair(kp_top, kp_bottom, NKP)

    bn = 512
    grid = (3, B // bn)
    ov, ovs, obb, okp = pl.pallas_call(
        _mm_kernel,
        grid=grid,
        in_specs=[
            pl.BlockSpec((KC, bn), lambda p, b: (p, b)),
            pl.BlockSpec((KC, V), lambda p, b: (0, 0)),
            pl.BlockSpec((KC, VSUB), lambda p, b: (0, 0)),
            pl.BlockSpec((KC, 2 * NBB), lambda p, b: (0, 0)),
            pl.BlockSpec((KC, 2 * NKP), lambda p, b: (0, 0)),
        ],
        out_specs=[
            pl.BlockSpec((V, bn), lambda p, b: (p, b)),
            pl.BlockSpec((VSUB, bn), lambda p, b: (p, b)),
            pl.BlockSpec((2 * NBB, bn), lambda p, b: (p, b)),
            pl.BlockSpec((2 * NKP, bn), lambda p, b: (p, b)),
        ],
        out_shape=[
            jax.ShapeDtypeStruct((3 * V, B), jnp.float32),
            jax.ShapeDtypeStruct((3 * VSUB, B), jnp.float32),
            jax.ShapeDtypeStruct((3 * 2 * NBB, B), jnp.float32),
            jax.ShapeDtypeStruct((3 * 2 * NKP, B), jnp.float32),
        ],
    )(w3, xv, xvs, xbb, xkp)

    v_out = jnp.transpose(ov.reshape(3, V, B), (2, 1, 0))
    vs_out = jnp.transpose(ovs.reshape(3, VSUB, B), (2, 1, 0))
    bbox3d = jnp.transpose(obb.reshape(3, 2 * NBB, B), (2, 1, 0))
    kp3d = jnp.transpose(okp.reshape(3, 2 * NKP, B), (2, 1, 0))
    return v_out, vs_out, bbox3d, kp3d
